# Initial kernel scaffold; baseline (speedup 1.0000x reference)
import jax
import jax.numpy as jnp
from jax.experimental import pallas as pl


def kernel(enc_output_, x, W_s_mono_w, W_s_mono_b, W_h_mono_w, v_mono_w, g_mono, r_mono, W_s_chunk_w, W_s_chunk_b, W_h_chunk_w, v_chunk_w, L_sy_w, L_gy_w, L_gy_b, L_yy_w, L_yy_b, L_ys_w, L_ss_w, L_gs_w, L_gs_b):
    raise NotImplementedError("write your pallas kernel here")


# speedup vs baseline: 4.4495x; 4.4495x over previous
import jax
import jax.numpy as jnp
from jax.experimental import pallas as pl
from jax.experimental.pallas import tpu as pltpu

H = 512
C = 1000
CPAD = 1024
SEQ = 4096
WIN = 4
MAX_STEP = 8
BLK = 128
NEG = -1e30


def _fused(enc_ref, lys_ref,
           wsm_ref, wsmb_ref, whm_ref, vm_ref, gm_ref, rm_ref,
           wsc_ref, wscb_ref, whc_ref, vc_ref,
           lsy_ref, lgy_ref, lgyb_ref, lyy_ref, lyyb_ref,
           lss_ref, lgs_ref, lgsb_ref,
           out_ref,
           blk_vmem, win_vmem, row_vmem, sem_blk, sem_win, sem_row):
    f32 = jnp.float32

    def dott(a, b_ref_or_val):
        return jax.lax.dot_general(a, b_ref_or_val, (((1,), (1,)), ((), ())),
                                   preferred_element_type=f32)

    v_mono = vm_ref[...]
    v_norm_mono = gm_ref[0, 0] * jax.lax.rsqrt(jnp.sum(v_mono * v_mono))
    r_mono = rm_ref[0, 0]

    def step_body(step, carry):
        s, c, cur_idx, alive = carry
        bias_mono = dott(s, wsm_ref[...]) + wsmb_ref[...]

        def scan_cond(sc):
            pos, _fidx, found = sc
            return jnp.logical_and(
                jnp.logical_and(alive, jnp.logical_not(found)), pos < SEQ)

        def scan_body(sc):
            pos, fidx, _found = sc
            start = pl.multiple_of(jnp.minimum(pos, SEQ - BLK) & ~7, 8)
            cp = pltpu.make_async_copy(
                enc_ref.at[pl.ds(start, BLK), :], blk_vmem, sem_blk)
            cp.start()
            cp.wait()
            t = jnp.tanh(dott(blk_vmem[...], whm_ref[...]) + bias_mono)
            e = v_norm_mono * dott(t, v_mono) + r_mono
            p = jax.nn.sigmoid(e)
            jg = start + jax.lax.broadcasted_iota(jnp.int32, (BLK, 1), 0)
            ok = jnp.logical_and(p >= 0.5, jg >= cur_idx)
            cand = jnp.where(ok, jg, SEQ)
            m = jnp.min(cand)
            found_new = m < SEQ
            fidx_new = jnp.where(found_new, m, fidx)
            return (start + BLK, fidx_new, found_new)

        _pos, fidx, found = jax.lax.while_loop(
            scan_cond, scan_body, (cur_idx, jnp.int32(0), jnp.bool_(False)))
        any_fired = jnp.logical_and(found, alive)
        fired_index = jnp.where(any_fired, fidx, 0)

        wstart = jnp.maximum(fired_index - (WIN - 1), 0)
        wa = pl.multiple_of(jnp.minimum(wstart & ~7, SEQ - 16), 8)
        cpw = pltpu.make_async_copy(
            enc_ref.at[pl.ds(wa, 16), :], win_vmem, sem_win)
        cpw.start()
        cpw.wait()
        bias_chunk = dott(s, wsc_ref[...]) + wscb_ref[...]
        tw = jnp.tanh(dott(win_vmem[...], whc_ref[...]) + bias_chunk)
        ec = dott(tw, vc_ref[...])
        idx_g = wa + jax.lax.broadcasted_iota(jnp.int32, (16, 1), 0)
        in_win = jnp.logical_and(idx_g >= wstart, idx_g < wstart + WIN)
        validf = jnp.logical_and(in_win, idx_g <= fired_index)
        en = jnp.where(in_win, ec * validf.astype(f32), NEG)
        b = jnp.exp(en - jnp.max(en))
        beta = b / jnp.sum(b)
        coef = jnp.where(validf, beta, 0.0)
        wm = jnp.where(validf, win_vmem[...], 0.0)
        context = jnp.sum(wm * coef, axis=0, keepdims=True)

        h = jnp.tanh(dott(context, lgy_ref[...]) + lgyb_ref[...]
                     + dott(s, lsy_ref[...]))
        y = dott(h, lyy_ref[...]) + lyyb_ref[...]
        out_ref[pl.ds(step, 1), :] = y
        ymax = jnp.max(y)
        col = jax.lax.broadcasted_iota(jnp.int32, (1, CPAD), 1)
        tok = jnp.min(jnp.where(y == ymax, col, CPAD))

        ta = pl.multiple_of(jnp.minimum(tok & ~7, C - 8), 8)
        cpr = pltpu.make_async_copy(
            lys_ref.at[pl.ds(ta, 8), :], row_vmem, sem_row)
        cpr.start()
        cpr.wait()
        rsel = (ta + jax.lax.broadcasted_iota(jnp.int32, (8, 1), 0)
                ) == tok
        ys_row = jnp.sum(jnp.where(rsel, row_vmem[...], 0.0),
                         axis=0, keepdims=True)
        rec = (ys_row + dott(s, lss_ref[...])
               + dott(context, lgs_ref[...]) + lgsb_ref[...])
        ig = rec[:, 0:H]
        fg = rec[:, H:2 * H]
        gg = rec[:, 2 * H:3 * H]
        og = rec[:, 3 * H:4 * H]
        c_new = jax.nn.sigmoid(fg) * c + jax.nn.sigmoid(ig) * jnp.tanh(gg)
        s_new = jax.nn.sigmoid(og) * jnp.tanh(c_new)
        s = jnp.where(any_fired, s_new, s)
        c = jnp.where(any_fired, c_new, c)
        return (s, c, fired_index, any_fired)

    s0 = jnp.zeros((1, H), f32)
    c0 = jnp.zeros((1, H), f32)
    jax.lax.fori_loop(0, MAX_STEP, step_body,
                      (s0, c0, jnp.int32(0), jnp.bool_(True)))


def kernel(enc_output_, x, W_s_mono_w, W_s_mono_b, W_h_mono_w, v_mono_w,
           g_mono, r_mono, W_s_chunk_w, W_s_chunk_b, W_h_chunk_w, v_chunk_w,
           L_sy_w, L_gy_w, L_gy_b, L_yy_w, L_yy_b, L_ys_w, L_ss_w, L_gs_w,
           L_gs_b):
    del x
    f32 = jnp.float32
    enc = enc_output_.reshape(SEQ, 2 * H)
    lyy_pad = jnp.zeros((CPAD, H), f32).at[:C, :].set(L_yy_w)
    lyyb_pad = jnp.full((1, CPAD), NEG, f32).at[0, :C].set(L_yy_b)

    vmem = pl.BlockSpec(memory_space=pltpu.VMEM)
    anymem = pl.BlockSpec(memory_space=pltpu.MemorySpace.HBM)

    out = pl.pallas_call(
        _fused,
        out_shape=jax.ShapeDtypeStruct((MAX_STEP, CPAD), f32),
        in_specs=[anymem, anymem] + [vmem] * 18,
        out_specs=vmem,
        scratch_shapes=[
            pltpu.VMEM((BLK, 2 * H), f32),
            pltpu.VMEM((16, 2 * H), f32),
            pltpu.VMEM((8, 4 * H), f32),
            pltpu.SemaphoreType.DMA,
            pltpu.SemaphoreType.DMA,
            pltpu.SemaphoreType.DMA,
        ],
        compiler_params=pltpu.CompilerParams(
            vmem_limit_bytes=100 * 1024 * 1024,
        ),
    )(enc, L_ys_w,
      W_s_mono_w, W_s_mono_b.reshape(1, 2 * H), W_h_mono_w, v_mono_w,
      g_mono.reshape(1, 1), r_mono.reshape(1, 1),
      W_s_chunk_w, W_s_chunk_b.reshape(1, 2 * H), W_h_chunk_w, v_chunk_w,
      L_sy_w, L_gy_w, L_gy_b.reshape(1, H), lyy_pad, lyyb_pad,
      L_ss_w, L_gs_w, L_gs_b.reshape(1, 4 * H))
    return out[:, :C]


# speedup vs baseline: 5.6396x; 1.2675x over previous
import jax
import jax.numpy as jnp
from jax.experimental import pallas as pl
from jax.experimental.pallas import tpu as pltpu

H = 512
C = 1000
CPAD = 1024
SEQ = 4096
WIN = 4
MAX_STEP = 8
BLK = 128
NEG = -1e30


def _blk_start(pos):
    return pl.multiple_of(jnp.minimum(pos, SEQ - BLK) & ~7, 8)


def _fused(enc_ref, lys_ref,
           wsm_ref, wsmb_ref, whm_ref, vm_ref, gm_ref, rm_ref,
           wsc_ref, wscb_ref, whc_ref, vc_ref,
           lsy_ref, lgy_ref, lgyb_ref, lyy_ref, lyyb_ref,
           lss_ref, lgs_ref, lgsb_ref,
           out_ref,
           blk_vmem, win_vmem, row_vmem, sem_blk, sem_win, sem_row):
    f32 = jnp.float32

    def dott(a, b):
        return jax.lax.dot_general(a, b, (((1,), (1,)), ((), ())),
                                   preferred_element_type=f32)

    def blk_copy(start):
        return pltpu.make_async_copy(
            enc_ref.at[pl.ds(start, BLK), :], blk_vmem, sem_blk)

    v_mono = vm_ref[...]
    v_norm_mono = gm_ref[0, 0] * jax.lax.rsqrt(jnp.sum(v_mono * v_mono))
    r_mono = rm_ref[0, 0]

    def energies(start, cur_idx, bias_mono):
        t = jnp.tanh(dott(blk_vmem[...], whm_ref[...]) + bias_mono)
        e = v_norm_mono * dott(t, v_mono) + r_mono
        p = jax.nn.sigmoid(e)
        jg = start + jax.lax.broadcasted_iota(jnp.int32, (BLK, 1), 0)
        ok = jnp.logical_and(p >= 0.5, jg >= cur_idx)
        m = jnp.min(jnp.where(ok, jg, SEQ))
        return m < SEQ, m

    def step_body(step, carry):
        s, c, cur_idx, alive = carry
        start0 = _blk_start(cur_idx)
        bias_mono = dott(s, wsm_ref[...]) + wsmb_ref[...]
        blk_copy(start0).wait()
        found0, fidx0 = energies(start0, cur_idx, bias_mono)

        def scan_cond(sc):
            pos, _fidx, found = sc
            return jnp.logical_and(
                jnp.logical_and(alive, jnp.logical_not(found)), pos < SEQ)

        def scan_body(sc):
            pos, fidx, _found = sc
            start = _blk_start(pos)
            cp = blk_copy(start)
            cp.start()
            cp.wait()
            found_new, m = energies(start, cur_idx, bias_mono)
            fidx_new = jnp.where(found_new, m, fidx)
            return (start + BLK, fidx_new, found_new)

        _pos, fidx, found = jax.lax.while_loop(
            scan_cond, scan_body, (start0 + BLK, fidx0, found0))
        any_fired = jnp.logical_and(found, alive)
        fired_index = jnp.where(any_fired, fidx, 0)

        blk_copy(_blk_start(fired_index)).start()

        wstart = jnp.maximum(fired_index - (WIN - 1), 0)
        wa = pl.multiple_of(jnp.minimum(wstart & ~7, SEQ - 16), 8)
        cpw = pltpu.make_async_copy(
            enc_ref.at[pl.ds(wa, 16), :], win_vmem, sem_win)
        cpw.start()
        bias_chunk = dott(s, wsc_ref[...]) + wscb_ref[...]
        cpw.wait()
        tw = jnp.tanh(dott(win_vmem[...], whc_ref[...]) + bias_chunk)
        ec = dott(tw, vc_ref[...])
        idx_g = wa + jax.lax.broadcasted_iota(jnp.int32, (16, 1), 0)
        in_win = jnp.logical_and(idx_g >= wstart, idx_g < wstart + WIN)
        validf = jnp.logical_and(in_win, idx_g <= fired_index)
        en = jnp.where(in_win, ec * validf.astype(f32), NEG)
        b = jnp.exp(en - jnp.max(en))
        beta = b / jnp.sum(b)
        coef = jnp.where(validf, beta, 0.0)
        wm = jnp.where(validf, win_vmem[...], 0.0)
        context = jnp.sum(wm * coef, axis=0, keepdims=True)

        h = jnp.tanh(dott(context, lgy_ref[...]) + lgyb_ref[...]
                     + dott(s, lsy_ref[...]))
        y = dott(h, lyy_ref[...]) + lyyb_ref[...]
        out_ref[pl.ds(step, 1), :] = y
        ymax = jnp.max(y)
        col = jax.lax.broadcasted_iota(jnp.int32, (1, CPAD), 1)
        tok = jnp.min(jnp.where(y == ymax, col, CPAD))

        ta = pl.multiple_of(jnp.minimum(tok & ~7, C - 8), 8)
        cpr = pltpu.make_async_copy(
            lys_ref.at[pl.ds(ta, 8), :], row_vmem, sem_row)
        cpr.start()
        rec_mm = (dott(s, lss_ref[...]) + dott(context, lgs_ref[...])
                  + lgsb_ref[...])
        cpr.wait()
        rsel = (ta + jax.lax.broadcasted_iota(jnp.int32, (8, 1), 0)) == tok
        ys_row = jnp.sum(jnp.where(rsel, row_vmem[...], 0.0),
                         axis=0, keepdims=True)
        rec = ys_row + rec_mm
        ig = rec[:, 0:H]
        fg = rec[:, H:2 * H]
        gg = rec[:, 2 * H:3 * H]
        og = rec[:, 3 * H:4 * H]
        c_new = jax.nn.sigmoid(fg) * c + jax.nn.sigmoid(ig) * jnp.tanh(gg)
        s_new = jax.nn.sigmoid(og) * jnp.tanh(c_new)
        s = jnp.where(any_fired, s_new, s)
        c = jnp.where(any_fired, c_new, c)
        return (s, c, fired_index, any_fired)

    blk_copy(_blk_start(jnp.int32(0))).start()
    s0 = jnp.zeros((1, H), f32)
    c0 = jnp.zeros((1, H), f32)
    s, c, cur_idx, alive = jax.lax.fori_loop(
        0, MAX_STEP, step_body, (s0, c0, jnp.int32(0), jnp.bool_(True)))
    blk_copy(_blk_start(cur_idx)).wait()


def kernel(enc_output_, x, W_s_mono_w, W_s_mono_b, W_h_mono_w, v_mono_w,
           g_mono, r_mono, W_s_chunk_w, W_s_chunk_b, W_h_chunk_w, v_chunk_w,
           L_sy_w, L_gy_w, L_gy_b, L_yy_w, L_yy_b, L_ys_w, L_ss_w, L_gs_w,
           L_gs_b):
    del x
    f32 = jnp.float32
    enc = enc_output_.reshape(SEQ, 2 * H)
    lyy_pad = jnp.zeros((CPAD, H), f32).at[:C, :].set(L_yy_w)
    lyyb_pad = jnp.full((1, CPAD), NEG, f32).at[0, :C].set(L_yy_b)

    vmem = pl.BlockSpec(memory_space=pltpu.VMEM)
    anymem = pl.BlockSpec(memory_space=pltpu.MemorySpace.HBM)

    out = pl.pallas_call(
        _fused,
        out_shape=jax.ShapeDtypeStruct((MAX_STEP, CPAD), f32),
        in_specs=[anymem, anymem] + [vmem] * 18,
        out_specs=vmem,
        scratch_shapes=[
            pltpu.VMEM((BLK, 2 * H), f32),
            pltpu.VMEM((16, 2 * H), f32),
            pltpu.VMEM((8, 4 * H), f32),
            pltpu.SemaphoreType.DMA,
            pltpu.SemaphoreType.DMA,
            pltpu.SemaphoreType.DMA,
        ],
        compiler_params=pltpu.CompilerParams(
            vmem_limit_bytes=100 * 1024 * 1024,
        ),
    )(enc, L_ys_w,
      W_s_mono_w, W_s_mono_b.reshape(1, 2 * H), W_h_mono_w, v_mono_w,
      g_mono.reshape(1, 1), r_mono.reshape(1, 1),
      W_s_chunk_w, W_s_chunk_b.reshape(1, 2 * H), W_h_chunk_w, v_chunk_w,
      L_sy_w, L_gy_w, L_gy_b.reshape(1, H), lyy_pad, lyyb_pad,
      L_ss_w, L_gs_w, L_gs_b.reshape(1, 4 * H))
    return out[:, :C]


# speedup vs baseline: 5.8036x; 1.0291x over previous
import jax
import jax.numpy as jnp
from jax.experimental import pallas as pl
from jax.experimental.pallas import tpu as pltpu

H = 512
C = 1000
CPAD = 1024
SEQ = 4096
WIN = 4
MAX_STEP = 8
BLK = 64
NEG = -1e30


def _blk_start(pos):
    return pl.multiple_of(jnp.minimum(pos, SEQ - BLK) & ~7, 8)


def _fused(enc_ref, lys_ref,
           wsm_ref, wsmb_ref, whm_ref, vm_ref, gm_ref, rm_ref,
           wsc_ref, wscb_ref, whc_ref, vc_ref,
           lsy_ref, lgy_ref, lgyb_ref, lyy_ref, lyyb_ref,
           lss_ref, lgs_ref, lgsb_ref,
           out_ref,
           blk_vmem, win_vmem, row_vmem, sem_blk, sem_win, sem_row):
    f32 = jnp.float32

    def dott(a, b):
        return jax.lax.dot_general(a, b, (((1,), (1,)), ((), ())),
                                   preferred_element_type=f32)

    def blk_copy(start):
        return pltpu.make_async_copy(
            enc_ref.at[pl.ds(start, BLK), :], blk_vmem, sem_blk)

    v_mono = vm_ref[...]
    v_norm_mono = gm_ref[0, 0] * jax.lax.rsqrt(jnp.sum(v_mono * v_mono))
    r_mono = rm_ref[0, 0]

    def energies(start, cur_idx, bias_mono):
        t = jnp.tanh(dott(blk_vmem[...], whm_ref[...]) + bias_mono)
        e = v_norm_mono * dott(t, v_mono) + r_mono
        p = jax.nn.sigmoid(e)
        jg = start + jax.lax.broadcasted_iota(jnp.int32, (BLK, 1), 0)
        ok = jnp.logical_and(p >= 0.5, jg >= cur_idx)
        m = jnp.min(jnp.where(ok, jg, SEQ))
        return m < SEQ, m

    def step_body(step, carry):
        s, c, cur_idx, alive = carry
        start0 = _blk_start(cur_idx)
        bias_mono = dott(s, wsm_ref[...]) + wsmb_ref[...]
        blk_copy(start0).wait()
        found0, fidx0 = energies(start0, cur_idx, bias_mono)

        def scan_cond(sc):
            pos, _fidx, found = sc
            return jnp.logical_and(
                jnp.logical_and(alive, jnp.logical_not(found)), pos < SEQ)

        def scan_body(sc):
            pos, fidx, _found = sc
            start = _blk_start(pos)
            cp = blk_copy(start)
            cp.start()
            cp.wait()
            found_new, m = energies(start, cur_idx, bias_mono)
            fidx_new = jnp.where(found_new, m, fidx)
            return (start + BLK, fidx_new, found_new)

        _pos, fidx, found = jax.lax.while_loop(
            scan_cond, scan_body, (start0 + BLK, fidx0, found0))
        any_fired = jnp.logical_and(found, alive)
        fired_index = jnp.where(any_fired, fidx, 0)

        blk_copy(_blk_start(fired_index)).start()

        wstart = jnp.maximum(fired_index - (WIN - 1), 0)
        wa = pl.multiple_of(jnp.minimum(wstart & ~7, SEQ - 16), 8)
        cpw = pltpu.make_async_copy(
            enc_ref.at[pl.ds(wa, 16), :], win_vmem, sem_win)
        cpw.start()
        bias_chunk = dott(s, wsc_ref[...]) + wscb_ref[...]
        cpw.wait()
        tw = jnp.tanh(dott(win_vmem[...], whc_ref[...]) + bias_chunk)
        ec = dott(tw, vc_ref[...])
        idx_g = wa + jax.lax.broadcasted_iota(jnp.int32, (16, 1), 0)
        in_win = jnp.logical_and(idx_g >= wstart, idx_g < wstart + WIN)
        validf = jnp.logical_and(in_win, idx_g <= fired_index)
        en = jnp.where(in_win, ec * validf.astype(f32), NEG)
        b = jnp.exp(en - jnp.max(en))
        beta = b / jnp.sum(b)
        coef = jnp.where(validf, beta, 0.0)
        wm = jnp.where(validf, win_vmem[...], 0.0)
        context = jnp.sum(wm * coef, axis=0, keepdims=True)

        h = jnp.tanh(dott(context, lgy_ref[...]) + lgyb_ref[...]
                     + dott(s, lsy_ref[...]))
        y = dott(h, lyy_ref[...]) + lyyb_ref[...]
        out_ref[pl.ds(step, 1), :] = y
        ymax = jnp.max(y)
        col = jax.lax.broadcasted_iota(jnp.int32, (1, CPAD), 1)
        tok = jnp.min(jnp.where(y == ymax, col, CPAD))

        ta = pl.multiple_of(jnp.minimum(tok & ~7, C - 8), 8)
        cpr = pltpu.make_async_copy(
            lys_ref.at[pl.ds(ta, 8), :], row_vmem, sem_row)
        cpr.start()
        rec_mm = (dott(s, lss_ref[...]) + dott(context, lgs_ref[...])
                  + lgsb_ref[...])
        cpr.wait()
        rsel = (ta + jax.lax.broadcasted_iota(jnp.int32, (8, 1), 0)) == tok
        ys_row = jnp.sum(jnp.where(rsel, row_vmem[...], 0.0),
                         axis=0, keepdims=True)
        rec = ys_row + rec_mm
        ig = rec[:, 0:H]
        fg = rec[:, H:2 * H]
        gg = rec[:, 2 * H:3 * H]
        og = rec[:, 3 * H:4 * H]
        c_new = jax.nn.sigmoid(fg) * c + jax.nn.sigmoid(ig) * jnp.tanh(gg)
        s_new = jax.nn.sigmoid(og) * jnp.tanh(c_new)
        s = jnp.where(any_fired, s_new, s)
        c = jnp.where(any_fired, c_new, c)
        return (s, c, fired_index, any_fired)

    blk_copy(_blk_start(jnp.int32(0))).start()
    s0 = jnp.zeros((1, H), f32)
    c0 = jnp.zeros((1, H), f32)
    s, c, cur_idx, alive = jax.lax.fori_loop(
        0, MAX_STEP, step_body, (s0, c0, jnp.int32(0), jnp.bool_(True)))
    blk_copy(_blk_start(cur_idx)).wait()


def kernel(enc_output_, x, W_s_mono_w, W_s_mono_b, W_h_mono_w, v_mono_w,
           g_mono, r_mono, W_s_chunk_w, W_s_chunk_b, W_h_chunk_w, v_chunk_w,
           L_sy_w, L_gy_w, L_gy_b, L_yy_w, L_yy_b, L_ys_w, L_ss_w, L_gs_w,
           L_gs_b):
    del x
    f32 = jnp.float32
    enc = enc_output_.reshape(SEQ, 2 * H)
    lyy_pad = jnp.zeros((CPAD, H), f32).at[:C, :].set(L_yy_w)
    lyyb_pad = jnp.full((1, CPAD), NEG, f32).at[0, :C].set(L_yy_b)

    vmem = pl.BlockSpec(memory_space=pltpu.VMEM)
    anymem = pl.BlockSpec(memory_space=pltpu.MemorySpace.HBM)

    out = pl.pallas_call(
        _fused,
        out_shape=jax.ShapeDtypeStruct((MAX_STEP, CPAD), f32),
        in_specs=[anymem, anymem] + [vmem] * 18,
        out_specs=vmem,
        scratch_shapes=[
            pltpu.VMEM((BLK, 2 * H), f32),
            pltpu.VMEM((16, 2 * H), f32),
            pltpu.VMEM((8, 4 * H), f32),
            pltpu.SemaphoreType.DMA,
            pltpu.SemaphoreType.DMA,
            pltpu.SemaphoreType.DMA,
        ],
        compiler_params=pltpu.CompilerParams(
            vmem_limit_bytes=100 * 1024 * 1024,
        ),
    )(enc, L_ys_w,
      W_s_mono_w, W_s_mono_b.reshape(1, 2 * H), W_h_mono_w, v_mono_w,
      g_mono.reshape(1, 1), r_mono.reshape(1, 1),
      W_s_chunk_w, W_s_chunk_b.reshape(1, 2 * H), W_h_chunk_w, v_chunk_w,
      L_sy_w, L_gy_w, L_gy_b.reshape(1, H), lyy_pad, lyyb_pad,
      L_ss_w, L_gs_w, L_gs_b.reshape(1, 4 * H))
    return out[:, :C]
